# baseline (speedup 1.0000x reference)
import functools

import jax
import jax.numpy as jnp
from jax import lax
from jax.experimental import pallas as pl
from jax.experimental.pallas import tpu as pltpu
from jax.experimental.pallas import tpu_sc as plsc

_LANES = 16
_GATHER_CHUNK = 128


def kernel(x, idx, embedding_weight):
    B, D = x.shape
    info = plsc.get_sparse_core_info()
    nc, ns = info.num_cores, info.num_subcores
    nw = nc * ns
    b_per_w = B // nw
    n_chunks = b_per_w // _GATHER_CHUNK
    d_vecs = D // _LANES

    mesh = plsc.VectorSubcoreMesh(core_axis_name="c", subcore_axis_name="s")

    @functools.partial(
        pl.kernel,
        mesh=mesh,
        out_type=jax.ShapeDtypeStruct((B, D), jnp.float32),
        compiler_params=pltpu.CompilerParams(use_tc_tiling_on_sc=False),
        scratch_types=[
            pltpu.VMEM((b_per_w,), jnp.int32),
            pltpu.VMEM((b_per_w, D), jnp.float32),
            pltpu.VMEM((b_per_w, D), jnp.float32),
            pltpu.SemaphoreType.DMA,
        ],
    )
    def _masked(x_hbm, idx_hbm, table_hbm, out_hbm, idx_v, rows_v, x_v, sem):
        wid = lax.axis_index("s") * nc + lax.axis_index("c")
        base = wid * b_per_w

        pltpu.sync_copy(idx_hbm.at[pl.ds(base, b_per_w)], idx_v)
        copies = []
        for k in range(n_chunks):
            copies.append(
                pltpu.async_copy(
                    table_hbm.at[idx_v.at[pl.ds(k * _GATHER_CHUNK, _GATHER_CHUNK)]],
                    rows_v.at[pl.ds(k * _GATHER_CHUNK, _GATHER_CHUNK)],
                    sem,
                )
            )
        pltpu.sync_copy(x_hbm.at[pl.ds(base, b_per_w)], x_v)
        for c in copies:
            c.wait()

        def body(r, _):
            for d in range(d_vecs):
                m = rows_v[r, pl.ds(d * _LANES, _LANES)]
                xv = x_v[r, pl.ds(d * _LANES, _LANES)]
                rows_v[r, pl.ds(d * _LANES, _LANES)] = xv / (1.0 + jnp.exp(-m))
            return 0

        lax.fori_loop(0, b_per_w, body, 0)

        pltpu.sync_copy(rows_v, out_hbm.at[pl.ds(base, b_per_w)])

    return _masked(x, idx, embedding_weight)


# speedup vs baseline: 1.4710x; 1.4710x over previous
import functools

import jax
import jax.numpy as jnp
from jax import lax
from jax.experimental import pallas as pl
from jax.experimental.pallas import tpu as pltpu
from jax.experimental.pallas import tpu_sc as plsc

_LANES = 16


def kernel(x, idx, embedding_weight):
    B, D = x.shape
    info = plsc.get_sparse_core_info()
    nc, ns = info.num_cores, info.num_subcores
    nw = nc * ns
    b_per_w = B // nw
    d_vecs = D // _LANES
    n_groups = b_per_w // _LANES

    mesh = plsc.VectorSubcoreMesh(core_axis_name="c", subcore_axis_name="s")

    @functools.partial(
        pl.kernel,
        mesh=mesh,
        out_type=jax.ShapeDtypeStruct((B, D), jnp.float32),
        scratch_types=[
            pltpu.VMEM((b_per_w,), jnp.int32),
            pltpu.VMEM((2, _LANES, 8, D), jnp.float32),
            pltpu.VMEM((b_per_w, D), jnp.float32),
            pltpu.SemaphoreType.DMA,
            pltpu.SemaphoreType.DMA,
        ],
    )
    def _masked(x_hbm, idx_hbm, table_hbm, out_hbm, idx_v, blk_v, x_v, sem0, sem1):
        wid = lax.axis_index("s") * nc + lax.axis_index("c")
        base = wid * b_per_w
        sems = (sem0, sem1)

        pltpu.sync_copy(idx_hbm.at[pl.ds(base, b_per_w)], idx_v)
        pltpu.sync_copy(x_hbm.at[pl.ds(base, b_per_w)], x_v)

        def issue(g, buf):
            iv = idx_v[pl.ds(g * _LANES, _LANES)]
            for l in range(_LANES):
                j = iv[l]
                j8 = (j // 8) * 8
                pltpu.async_copy(
                    table_hbm.at[pl.ds(j8, 8)], blk_v.at[buf, l], sems[buf]
                )
            return iv

        def drain(buf):
            for l in range(_LANES):
                pltpu.make_async_copy(
                    table_hbm.at[pl.ds(0, 8)], blk_v.at[buf, l], sems[buf]
                ).wait()

        def compute(g, buf, iv):
            for l in range(_LANES):
                j = iv[l]
                r = j - (j // 8) * 8
                i = g * _LANES + l
                for d in range(d_vecs):
                    m = blk_v[buf, l, r, pl.ds(d * _LANES, _LANES)]
                    xv = x_v[i, pl.ds(d * _LANES, _LANES)]
                    x_v[i, pl.ds(d * _LANES, _LANES)] = xv / (1.0 + jnp.exp(-m))

        iv0 = issue(0, 0)
        iv1 = issue(1, 1)

        def body(k, carry):
            iv_a, iv_b = carry
            g = 2 * k
            drain(0)
            compute(g, 0, iv_a)
            iv_a2 = issue(g + 2, 0)
            drain(1)
            compute(g + 1, 1, iv_b)
            iv_b2 = issue(g + 3, 1)
            return iv_a2, iv_b2

        iv_a, iv_b = lax.fori_loop(0, n_groups // 2 - 1, body, (iv0, iv1))
        drain(0)
        compute(n_groups - 2, 0, iv_a)
        drain(1)
        compute(n_groups - 1, 1, iv_b)

        pltpu.sync_copy(x_v, out_hbm.at[pl.ds(base, b_per_w)])

    return _masked(x, idx, embedding_weight)
